# baseline (speedup 1.0000x reference)
import jax
import jax.numpy as jnp
from jax import lax
from jax.experimental import pallas as pl
from jax.experimental.pallas import tpu as pltpu
from jax.experimental.pallas import tpu_sc as plsc

N = 10000
E = 320000
D_IN = 128
HEADS = 8
TH = 128
OUT = 40

NP = 10240
ND = N
NC = 2
NS = 16
NW = NC * NS
RPT = NP // NS
TW = TH + 16

ESL = E + N

C1 = 56
NCH1 = 196
EPW1 = NCH1 * C1
EP1 = EPW1 * NW

C2 = 48
NCH2 = 220
EPW2 = NCH2 * C2
EP2 = EPW2 * NW

_HI = jax.lax.Precision.HIGHEST


def _dot(a, b):
    return jax.lax.dot_general(a, b, (((1,), (0,)), ((), ())),
                               precision=_HI, preferred_element_type=jnp.float32)


def _k1_body(x_ref, rw_ref, rb_ref, gw_ref, ac_ref,
             xws_ref, d2_ref):
    xp = _dot(x_ref[...], rw_ref[...]) + rb_ref[...]
    xw = _dot(xp, gw_ref[...])
    sd = _dot(xw, ac_ref[...])
    xws_ref[...] = jnp.concatenate([xw, sd[:, :16]], axis=1)
    d2_ref[...] = sd[:, 16:]


def _k2_body(op_ref, em_ref, gb_ref, s_ref, t_ref, gw_ref, ac_ref,
             xws_ref, d2_ref):
    un = op_ref[0] + op_ref[1]
    dex = _dot(un[:, TH:], em_ref[...])
    g = un[:, :TH] / (dex + 1e-16) + gb_ref[...]
    g = g * s_ref[...] + t_ref[...]
    h = jnp.where(g > 0, g, jnp.exp(g) - 1.0)
    xw = _dot(h, gw_ref[...])
    sd = _dot(xw, ac_ref[...])
    xws_ref[...] = jnp.concatenate([xw, sd[:, :16]], axis=1)
    d2_ref[...] = sd[:, 16:]


def _k3_body(op_ref, em_ref, gb_ref, s_ref, t_ref, wa_ref, wb_ref, b1_ref,
             hb_ref, a_ref, b_ref):
    un = op_ref[0] + op_ref[1]
    dex = _dot(un[:, TH:], em_ref[...])
    g = un[:, :TH] / (dex + 1e-16) + gb_ref[...]
    g = g * s_ref[...] + t_ref[...]
    h = jnp.where(g > 0, g, jnp.exp(g) - 1.0)
    hb_ref[...] = h
    a_ref[...] = _dot(h, wa_ref[...])
    b_ref[...] = _dot(h, wb_ref[...]) + b1_ref[...]


def _k4_body(hb_ref, ag_ref, w1_ref, b1_ref, s_ref, t_ref, w2_ref, b2_ref,
             out_ref):
    hs = hb_ref[...] + ag_ref[0] + ag_ref[1]
    c1 = _dot(hs, w1_ref[...]) + b1_ref[...]
    c1 = c1 * s_ref[...] + t_ref[...]
    c1 = jnp.maximum(c1, 0.0)
    lg = _dot(c1, w2_ref[...]) + b2_ref[...]
    m = jnp.max(lg, axis=1, keepdims=True)
    lse = m + jnp.log(jnp.sum(jnp.exp(lg - m), axis=1, keepdims=True))
    out_ref[...] = lg - lse


def _row_spec(rb, cols):
    return pl.BlockSpec((rb, cols), lambda i: (i, 0))


def _full_spec(shape):
    nd = len(shape)
    return pl.BlockSpec(shape, lambda i: (0,) * nd)


_RB = 1024
_GRID = NP // _RB


def _tc_call(body, in_specs, out_specs, out_shapes, args):
    return pl.pallas_call(
        body,
        grid=(_GRID,),
        in_specs=in_specs,
        out_specs=out_specs,
        out_shape=out_shapes,
    )(*args)


_MESH = plsc.VectorSubcoreMesh(core_axis_name="c", subcore_axis_name="s")
_SC_PARAMS = pltpu.CompilerParams(use_tc_tiling_on_sc=False,
                                  needs_layout_passes=False)


def _gat_edge_body(rowsl, colsl, xwsrc, dst2, outp,
                   ridx0, ridx1, ridx2, ridx3,
                   cidx0, cidx1, cidx2, cidx3,
                   xwsr0, xwsr1, xwsr2,
                   dcol0, dcol1, dcol2,
                   out_sp,
                   isem0, isem1, isem2, isem3,
                   gsem0, gsem1, gsem2):
    cid = lax.axis_index("c")
    sid = lax.axis_index("s")
    w = cid * NS + sid
    ridx = [ridx0, ridx1, ridx2, ridx3]
    cidx = [cidx0, cidx1, cidx2, cidx3]
    xwsr = [xwsr0, xwsr1, xwsr2]
    dcol = [dcol0, dcol1, dcol2]
    isem = [isem0, isem1, isem2, isem3]
    gsem = [gsem0, gsem1, gsem2]

    zrow = sid * RPT
    def _z(i, _):
        for k in range(TW // 16):
            xwsr0[i, pl.ds(k * 16, 16)] = jnp.zeros((16,), jnp.float32)
        return 0
    lax.fori_loop(0, C1, _z, 0)
    nfull = RPT // C1
    def _zs(i, _):
        pltpu.sync_copy(xwsr0, out_sp.at[pl.ds(zrow + i * C1, C1)])
        return 0
    lax.fori_loop(0, nfull, _zs, 0)
    rem = RPT - nfull * C1
    if rem:
        pltpu.sync_copy(xwsr0.at[pl.ds(0, rem)],
                        out_sp.at[pl.ds(zrow + nfull * C1, rem)])
    plsc.subcore_barrier()

    def fire_idx(c, sl):
        gb = (w * NCH1 + c) * C1
        pltpu.async_copy(rowsl.at[pl.ds(gb, C1)], ridx[sl], isem[sl])
        pltpu.async_copy(colsl.at[pl.ds(gb, C1)], cidx[sl], isem[sl])

    def wait_idx(c, sl):
        gb = (w * NCH1 + c) * C1
        pltpu.make_async_copy(rowsl.at[pl.ds(gb, C1)], ridx[sl], isem[sl]).wait()
        pltpu.make_async_copy(colsl.at[pl.ds(gb, C1)], cidx[sl], isem[sl]).wait()

    def fire_gather(sd, si):
        pltpu.async_copy(xwsrc.at[ridx[si]], xwsr[sd], gsem[sd])
        pltpu.async_copy(dst2.at[cidx[si]], dcol[sd], gsem[sd])

    def wait_gather(sd, si):
        pltpu.make_async_copy(xwsrc.at[ridx[si]], xwsr[sd], gsem[sd]).wait()
        pltpu.make_async_copy(dst2.at[cidx[si]], dcol[sd], gsem[sd]).wait()

    def fire_scatter(sd, si):
        pltpu.sync_copy(xwsr[sd], out_sp.at[cidx[si]], add=True)

    def compute(sl):
        xb, db = xwsr[sl], dcol[sl]
        def edge(i, _2):
            a = xb[i, pl.ds(TH, 16)] + db[i]
            lr = jnp.maximum(a, a * 0.2)
            pe = jnp.exp(lr)
            xb[i, pl.ds(TH, 16)] = pe
            for h in range(HEADS):
                ph = pe[h]
                blk = xb[i, pl.ds(h * 16, 16)]
                xb[i, pl.ds(h * 16, 16)] = blk * ph
            return 0
        lax.fori_loop(0, C1, edge, 0)

    def stage(c, r, fi, fg):
        if fi:
            fire_idx(c + 2, (r + 2) % 4)
        if fg:
            wait_idx(c + 1, (r + 1) % 4)
            fire_gather((r + 1) % 3, (r + 1) % 4)
        wait_gather(r % 3, r % 4)
        compute(r % 3)
        fire_scatter(r % 3, r % 4)

    fire_idx(0, 0)
    fire_idx(1, 1)
    wait_idx(0, 0)
    fire_gather(0, 0)
    stage(0, 0, True, True)
    stage(1, 1, True, True)
    def steady(it, _):
        for j in range(12):
            stage(2 + it * 12 + j, (2 + j) % 12, True, True)
        return 0
    lax.fori_loop(0, (NCH1 - 4) // 12, steady, 0)
    stage(NCH1 - 2, (NCH1 - 2) % 12, False, True)
    stage(NCH1 - 1, (NCH1 - 1) % 12, False, False)
    plsc.subcore_barrier()

    pltpu.sync_copy(out_sp.at[pl.ds(zrow, RPT)], outp.at[cid, pl.ds(zrow, RPT)])


def _gat_edge_pass(rowsl, colsl, xwsrc, dst2):
    f = pl.kernel(
        _gat_edge_body,
        out_type=jax.ShapeDtypeStruct((NC, NP, TW), jnp.float32),
        mesh=_MESH,
        scratch_types=(
            *([pltpu.VMEM((C1,), jnp.int32)] * 8),
            *([pltpu.VMEM((C1, TW), jnp.float32)] * 3),
            *([pltpu.VMEM((C1, 16), jnp.float32)] * 3),
            pltpu.VMEM_SHARED((NP, TW), jnp.float32),
            *([pltpu.SemaphoreType.DMA] * 7),
        ),
        compiler_params=_SC_PARAMS,
    )
    return f(rowsl, colsl, xwsrc, dst2)


def _scorer_body(rowp, colp, tpad, abuf_h, bbuf_h, hb_h, w2c_h, sb2v_h,
                 lgw_out, aggp,
                 ridx0, ridx1, ridx2, ridx3,
                 cidx0, cidx1, cidx2, cidx3,
                 tbuf0, tbuf1, tbuf2, tbuf3,
                 arow0, arow1, arow2,
                 bcol0, bcol1, bcol2,
                 hcol0, hcol1, hcol2,
                 lgw0, lgw1, lgw2,
                 sidx0, sidx1, sidx2,
                 w2f, sb2b, agg_sp,
                 isem0, isem1, isem2, isem3,
                 gsem0, gsem1, gsem2):
    cid = lax.axis_index("c")
    sid = lax.axis_index("s")
    w = cid * NS + sid
    ridx = [ridx0, ridx1, ridx2, ridx3]
    cidx = [cidx0, cidx1, cidx2, cidx3]
    tbuf = [tbuf0, tbuf1, tbuf2, tbuf3]
    arow = [arow0, arow1, arow2]
    bcol = [bcol0, bcol1, bcol2]
    hcol = [hcol0, hcol1, hcol2]
    lgw = [lgw0, lgw1, lgw2]
    sidx = [sidx0, sidx1, sidx2]
    isem = [isem0, isem1, isem2, isem3]
    gsem = [gsem0, gsem1, gsem2]

    zrow = sid * RPT
    def _z(i, _):
        for k in range(8):
            hcol0[i, pl.ds(k * 16, 16)] = jnp.zeros((16,), jnp.float32)
        return 0
    lax.fori_loop(0, C2, _z, 0)
    nfull = RPT // C2
    def _zs(i, _):
        pltpu.sync_copy(hcol0, agg_sp.at[pl.ds(zrow + i * C2, C2)])
        return 0
    lax.fori_loop(0, nfull, _zs, 0)
    rem = RPT - nfull * C2
    if rem:
        pltpu.sync_copy(hcol0.at[pl.ds(0, rem)],
                        agg_sp.at[pl.ds(zrow + nfull * C2, rem)])
    pltpu.sync_copy(w2c_h, w2f)
    pltpu.sync_copy(sb2v_h, sb2b)
    plsc.subcore_barrier()

    def fire_idx(c, sl):
        gb = (w * NCH2 + c) * C2
        pltpu.async_copy(rowp.at[pl.ds(gb, C2)], ridx[sl], isem[sl])
        pltpu.async_copy(colp.at[pl.ds(gb, C2)], cidx[sl], isem[sl])
        pltpu.async_copy(tpad.at[pl.ds(gb, C2)], tbuf[sl], isem[sl])

    def wait_idx(c, sl):
        gb = (w * NCH2 + c) * C2
        pltpu.make_async_copy(rowp.at[pl.ds(gb, C2)], ridx[sl], isem[sl]).wait()
        pltpu.make_async_copy(colp.at[pl.ds(gb, C2)], cidx[sl], isem[sl]).wait()
        pltpu.make_async_copy(tpad.at[pl.ds(gb, C2)], tbuf[sl], isem[sl]).wait()

    def fire_gather(sd, si):
        pltpu.async_copy(abuf_h.at[ridx[si]], arow[sd], gsem[sd])
        pltpu.async_copy(bbuf_h.at[cidx[si]], bcol[sd], gsem[sd])
        pltpu.async_copy(hb_h.at[cidx[si]], hcol[sd], gsem[sd])

    def wait_gather(sd, si):
        pltpu.make_async_copy(abuf_h.at[ridx[si]], arow[sd], gsem[sd]).wait()
        pltpu.make_async_copy(bbuf_h.at[cidx[si]], bcol[sd], gsem[sd]).wait()
        pltpu.make_async_copy(hb_h.at[cidx[si]], hcol[sd], gsem[sd]).wait()

    def fire_scatter(c, sd):
        base = (w * NCH2 + c) * 96
        pltpu.sync_copy(lgw[sd], lgw_out.at[pl.ds(base, 96)])
        pltpu.sync_copy(hcol[sd], agg_sp.at[sidx[sd]], add=True)

    iota = lax.iota(jnp.int32, 16)

    def compute(sd, si):
        ar, bc = arow[sd], bcol[sd]
        lb, sx = lgw[sd], sidx[sd]
        tb, rx = tbuf[si], ridx[si]
        sb2 = sb2b[...]
        def group(j, _2):
            base16 = j * 16
            eidx = base16 + iota
            def kbody(k, acc):
                kv = jnp.full((16,), k, jnp.int32)
                av = plsc.load_gather(ar, [eidx, kv])
                bv = plsc.load_gather(bc, [eidx, kv])
                w2v = plsc.load_gather(w2f, [kv])
                return acc + jnp.maximum(av + bv, 0.0) * w2v
            acc = lax.fori_loop(0, 64, kbody, sb2, unroll=8)
            tv = tb[pl.ds(base16, 16)]
            rv = rx[pl.ds(base16, 16)]
            m = acc > tv
            lb[pl.ds(base16, 16)] = acc
            lb[pl.ds(C2 + base16, 16)] = jnp.where(m, 1.0, 0.0)
            sx[pl.ds(base16, 16)] = jnp.where(m, rv, ND)
            return 0
        lax.fori_loop(0, C2 // 16, group, 0)

    def stage(c, r, fi, fg):
        if fi:
            fire_idx(c + 2, (r + 2) % 4)
        if fg:
            wait_idx(c + 1, (r + 1) % 4)
            fire_gather((r + 1) % 3, (r + 1) % 4)
        wait_gather(r % 3, r % 4)
        compute(r % 3, r % 4)
        fire_scatter(c, r % 3)

    fire_idx(0, 0)
    fire_idx(1, 1)
    wait_idx(0, 0)
    fire_gather(0, 0)
    stage(0, 0, True, True)
    stage(1, 1, True, True)
    def steady(it, _):
        for j in range(12):
            stage(2 + it * 12 + j, (2 + j) % 12, True, True)
        return 0
    lax.fori_loop(0, (NCH2 - 4) // 12, steady, 0)
    stage(NCH2 - 2, (NCH2 - 2) % 12, False, True)
    stage(NCH2 - 1, (NCH2 - 1) % 12, False, False)
    plsc.subcore_barrier()

    pltpu.sync_copy(agg_sp.at[pl.ds(zrow, RPT)], aggp.at[cid, pl.ds(zrow, RPT)])


def _scorer_pass(rowp, colp, tpad, abuf, bbuf, hb, w2c, sb2v):
    f = pl.kernel(
        _scorer_body,
        out_type=(
            jax.ShapeDtypeStruct((NW * NCH2 * 96,), jnp.float32),
            jax.ShapeDtypeStruct((NC, NP, TH), jnp.float32),
        ),
        mesh=_MESH,
        scratch_types=(
            *([pltpu.VMEM((C2,), jnp.int32)] * 8),
            *([pltpu.VMEM((C2,), jnp.float32)] * 4),
            *([pltpu.VMEM((C2, 64), jnp.float32)] * 6),
            *([pltpu.VMEM((C2, TH), jnp.float32)] * 3),
            *([pltpu.VMEM((2 * C2,), jnp.float32)] * 3),
            *([pltpu.VMEM((C2,), jnp.int32)] * 3),
            pltpu.VMEM((64,), jnp.float32),
            pltpu.VMEM((16,), jnp.float32),
            pltpu.VMEM_SHARED((NP, TH), jnp.float32),
            *([pltpu.SemaphoreType.DMA] * 7),
        ),
        compiler_params=_SC_PARAMS,
    )
    return f(rowp, colp, tpad, abuf, bbuf, hb, w2c, sb2v)


def _kernel_impl(x, edge_index, params, consts):
    p = params
    row = edge_index[0]
    col = edge_index[1]
    (a1cat, a2cat, em, s1, t1, s2, t2, cs, ct, w2p, b2p, tfix) = consts

    sl = jnp.arange(N, dtype=jnp.int32)
    padE1 = jnp.full((EP1 - ESL,), ND, jnp.int32)
    rowsl = jnp.concatenate([row, sl, padE1])
    colsl = jnp.concatenate([col, sl, padE1])
    padE2 = jnp.full((EP2 - E,), ND, jnp.int32)
    rowp = jnp.concatenate([row, padE2])
    colp = jnp.concatenate([col, padE2])
    tpad = jnp.concatenate([tfix, jnp.full((EP2 - E,), 1e30, jnp.float32)])

    xpad = jnp.zeros((NP, D_IN), jnp.float32).at[:N].set(x)

    rb2 = p["res_b"].reshape(1, TH)
    g1b = p["g1_b"].reshape(1, TH)
    g2b = p["g2_b"].reshape(1, TH)

    xws1, d2a = _tc_call(
        _k1_body,
        [_row_spec(_RB, D_IN), _full_spec((D_IN, TH)), _full_spec((1, TH)),
         _full_spec((TH, TH)), _full_spec((TH, 32))],
        [_row_spec(_RB, TW), _row_spec(_RB, 16)],
        [jax.ShapeDtypeStruct((NP, TW), jnp.float32),
         jax.ShapeDtypeStruct((NP, 16), jnp.float32)],
        [xpad, p["res_W"], rb2, p["g1_W"], a1cat],
    )

    outp1 = _gat_edge_pass(rowsl, colsl, xws1, d2a)

    xws2, d2b = _tc_call(
        _k2_body,
        [pl.BlockSpec((NC, _RB, TW), lambda i: (0, i, 0)),
         _full_spec((16, TH)), _full_spec((1, TH)), _full_spec((1, TH)),
         _full_spec((1, TH)), _full_spec((TH, TH)), _full_spec((TH, 32))],
        [_row_spec(_RB, TW), _row_spec(_RB, 16)],
        [jax.ShapeDtypeStruct((NP, TW), jnp.float32),
         jax.ShapeDtypeStruct((NP, 16), jnp.float32)],
        [outp1, em, g1b, s1.reshape(1, TH), t1.reshape(1, TH),
         p["g2_W"], a2cat],
    )

    outp2 = _gat_edge_pass(rowsl, colsl, xws2, d2b)

    hb, abuf, bbuf = _tc_call(
        _k3_body,
        [pl.BlockSpec((NC, _RB, TW), lambda i: (0, i, 0)),
         _full_spec((16, TH)), _full_spec((1, TH)), _full_spec((1, TH)),
         _full_spec((1, TH)), _full_spec((TH, 64)), _full_spec((TH, 64)),
         _full_spec((1, 64))],
        [_row_spec(_RB, TH), _row_spec(_RB, 64), _row_spec(_RB, 64)],
        [jax.ShapeDtypeStruct((NP, TH), jnp.float32),
         jax.ShapeDtypeStruct((NP, 64), jnp.float32),
         jax.ShapeDtypeStruct((NP, 64), jnp.float32)],
        [outp2, em, g2b, s2.reshape(1, TH), t2.reshape(1, TH),
         p["s_W1"][:TH], p["s_W1"][TH:], p["s_b1"].reshape(1, 64)],
    )

    w2c = p["s_W2"][:, 0]
    sb2v = jnp.full((16,), p["s_b2"][0], jnp.float32)
    lgw, aggp = _scorer_pass(rowp, colp, tpad, abuf, bbuf, hb, w2c, sb2v)

    (out,) = _tc_call(
        _k4_body,
        [_row_spec(_RB, TH),
         pl.BlockSpec((NC, _RB, TH), lambda i: (0, i, 0)),
         _full_spec((TH, 64)), _full_spec((1, 64)), _full_spec((1, 64)),
         _full_spec((1, 64)), _full_spec((64, TH)), _full_spec((1, TH))],
        [_row_spec(_RB, TH)],
        [jax.ShapeDtypeStruct((NP, TH), jnp.float32)],
        [hb, aggp, p["c_W1"], p["c_b1"].reshape(1, 64), cs.reshape(1, 64),
         ct.reshape(1, 64), w2p, b2p.reshape(1, TH)],
    )

    lgw2 = lgw.reshape(NW * NCH2, 2, C2)
    logits = lgw2[:, 0, :].reshape(EP2)[:E]
    weights = lgw2[:, 1, :].reshape(EP2)[:E]
    return out[:N, :OUT], weights, logits


def _make_consts(params):
    p = params

    def acat(a_s, a_d):
        eye = jnp.eye(HEADS, dtype=jnp.float32)
        ms = (a_s[:, :, None] * eye[:, None, :]).reshape(TH, HEADS)
        md = (a_d[:, :, None] * eye[:, None, :]).reshape(TH, HEADS)
        return jnp.concatenate([ms, ms, md, md], axis=1)

    a1cat = acat(p["g1_as"], p["g1_ad"])
    a2cat = acat(p["g2_as"], p["g2_ad"])
    em = jnp.concatenate(
        [jnp.kron(jnp.eye(HEADS, dtype=jnp.float32), jnp.ones((1, 16), jnp.float32)),
         jnp.zeros((8, TH), jnp.float32)], axis=0)

    def bnst(g, b, m, v):
        s = g / jnp.sqrt(v + 1e-5)
        return s, b - m * s

    s1, t1 = bnst(p["bn1_g"], p["bn1_b"], p["bn1_m"], p["bn1_v"])
    s2, t2 = bnst(p["bn2_g"], p["bn2_b"], p["bn2_m"], p["bn2_v"])
    cs, ct = bnst(p["cbn_g"], p["cbn_b"], p["cbn_m"], p["cbn_v"])

    w2p = jnp.zeros((64, TH), jnp.float32).at[:, :OUT].set(p["c_W2"])
    b2p = jnp.full((TH,), -1e30, jnp.float32).at[:OUT].set(p["c_b2"])

    u = jax.random.uniform(jax.random.key(42), (E, 2),
                           minval=1e-6, maxval=1.0 - 1e-6)
    g = -jnp.log(-jnp.log(u))
    tfix = g[:, 0] - g[:, 1]

    return (a1cat, a2cat, em, s1, t1, s2, t2, cs, ct, w2p, b2p, tfix)


@jax.jit
def kernel(x, edge_index, params):
    consts = _make_consts(params)
    return _kernel_impl(x, edge_index.astype(jnp.int32), params, consts)


# speedup vs baseline: 1.4043x; 1.4043x over previous
import jax
import jax.numpy as jnp
from jax import lax
from jax.experimental import pallas as pl
from jax.experimental.pallas import tpu as pltpu
from jax.experimental.pallas import tpu_sc as plsc

N = 10000
E = 320000
D_IN = 128
HEADS = 8
TH = 128
OUT = 40

NP = 10240
ND = N
NC = 2
NS = 16
NW = NC * NS
RPT = NP // NS
TW = TH + 16

ESL = E + N

C1 = 112
NCH1 = 96
EPW1 = NCH1 * C1
EP1 = EPW1 * NW

C2 = 80
NCH2 = 132
EPW2 = NCH2 * C2
EP2 = EPW2 * NW

_HI = jax.lax.Precision.HIGHEST


def _dot(a, b):
    return jax.lax.dot_general(a, b, (((1,), (0,)), ((), ())),
                               precision=_HI, preferred_element_type=jnp.float32)


def _k1_body(x_ref, rw_ref, rb_ref, gw_ref, ac_ref,
             xws_ref, d2_ref):
    xp = _dot(x_ref[...], rw_ref[...]) + rb_ref[...]
    xw = _dot(xp, gw_ref[...])
    sd = _dot(xw, ac_ref[...])
    xws_ref[...] = jnp.concatenate([xw, sd[:, :16]], axis=1)
    d2_ref[...] = sd[:, 16:]


def _k2_body(op_ref, em_ref, gb_ref, s_ref, t_ref, gw_ref, ac_ref,
             xws_ref, d2_ref):
    un = op_ref[0] + op_ref[1]
    dex = _dot(un[:, TH:], em_ref[...])
    g = un[:, :TH] / (dex + 1e-16) + gb_ref[...]
    g = g * s_ref[...] + t_ref[...]
    h = jnp.where(g > 0, g, jnp.exp(g) - 1.0)
    xw = _dot(h, gw_ref[...])
    sd = _dot(xw, ac_ref[...])
    xws_ref[...] = jnp.concatenate([xw, sd[:, :16]], axis=1)
    d2_ref[...] = sd[:, 16:]


def _k3_body(op_ref, em_ref, gb_ref, s_ref, t_ref, wa_ref, wb_ref, b1_ref,
             hb_ref, a_ref, b_ref):
    un = op_ref[0] + op_ref[1]
    dex = _dot(un[:, TH:], em_ref[...])
    g = un[:, :TH] / (dex + 1e-16) + gb_ref[...]
    g = g * s_ref[...] + t_ref[...]
    h = jnp.where(g > 0, g, jnp.exp(g) - 1.0)
    hb_ref[...] = h
    a_ref[...] = _dot(h, wa_ref[...])
    b_ref[...] = _dot(h, wb_ref[...]) + b1_ref[...]


def _k4_body(hb_ref, ag_ref, w1_ref, b1_ref, s_ref, t_ref, w2_ref, b2_ref,
             out_ref):
    hs = hb_ref[...] + ag_ref[0] + ag_ref[1]
    c1 = _dot(hs, w1_ref[...]) + b1_ref[...]
    c1 = c1 * s_ref[...] + t_ref[...]
    c1 = jnp.maximum(c1, 0.0)
    lg = _dot(c1, w2_ref[...]) + b2_ref[...]
    m = jnp.max(lg, axis=1, keepdims=True)
    lse = m + jnp.log(jnp.sum(jnp.exp(lg - m), axis=1, keepdims=True))
    out_ref[...] = lg - lse


def _row_spec(rb, cols):
    return pl.BlockSpec((rb, cols), lambda i: (i, 0))


def _full_spec(shape):
    nd = len(shape)
    return pl.BlockSpec(shape, lambda i: (0,) * nd)


_RB = 1024
_GRID = NP // _RB


def _tc_call(body, in_specs, out_specs, out_shapes, args):
    return pl.pallas_call(
        body,
        grid=(_GRID,),
        in_specs=in_specs,
        out_specs=out_specs,
        out_shape=out_shapes,
    )(*args)


_MESH = plsc.VectorSubcoreMesh(core_axis_name="c", subcore_axis_name="s")
_SC_PARAMS = pltpu.CompilerParams(use_tc_tiling_on_sc=False,
                                  needs_layout_passes=False)


def _gat_edge_body(rowsl, colsl, xwsrc, dst2, outp,
                   ridx0, ridx1, ridx2, ridx3,
                   cidx0, cidx1, cidx2, cidx3,
                   xwsr0, xwsr1,
                   dcol0, dcol1,
                   out_sp,
                   isem0, isem1, isem2, isem3,
                   gsem0, gsem1):
    cid = lax.axis_index("c")
    sid = lax.axis_index("s")
    w = cid * NS + sid
    ridx = [ridx0, ridx1, ridx2, ridx3]
    cidx = [cidx0, cidx1, cidx2, cidx3]
    xwsr = [xwsr0, xwsr1]
    dcol = [dcol0, dcol1]
    isem = [isem0, isem1, isem2, isem3]
    gsem = [gsem0, gsem1]

    zrow = sid * RPT
    def _z(i, _):
        for k in range(TW // 16):
            xwsr0[i, pl.ds(k * 16, 16)] = jnp.zeros((16,), jnp.float32)
        return 0
    lax.fori_loop(0, C1, _z, 0)
    nfull = RPT // C1
    def _zs(i, _):
        pltpu.sync_copy(xwsr0, out_sp.at[pl.ds(zrow + i * C1, C1)])
        return 0
    lax.fori_loop(0, nfull, _zs, 0)
    rem = RPT - nfull * C1
    if rem:
        pltpu.sync_copy(xwsr0.at[pl.ds(0, rem)],
                        out_sp.at[pl.ds(zrow + nfull * C1, rem)])
    plsc.subcore_barrier()

    def fire_idx(c, sl):
        gb = (w * NCH1 + c) * C1
        pltpu.async_copy(rowsl.at[pl.ds(gb, C1)], ridx[sl], isem[sl])
        pltpu.async_copy(colsl.at[pl.ds(gb, C1)], cidx[sl], isem[sl])

    def wait_idx(c, sl):
        gb = (w * NCH1 + c) * C1
        pltpu.make_async_copy(rowsl.at[pl.ds(gb, C1)], ridx[sl], isem[sl]).wait()
        pltpu.make_async_copy(colsl.at[pl.ds(gb, C1)], cidx[sl], isem[sl]).wait()

    def fire_gather(sd, si):
        pltpu.async_copy(xwsrc.at[ridx[si]], xwsr[sd], gsem[sd])
        pltpu.async_copy(dst2.at[cidx[si]], dcol[sd], gsem[sd])

    def wait_gather(sd, si):
        pltpu.make_async_copy(xwsrc.at[ridx[si]], xwsr[sd], gsem[sd]).wait()
        pltpu.make_async_copy(dst2.at[cidx[si]], dcol[sd], gsem[sd]).wait()

    def fire_scatter(sd, si):
        pltpu.sync_copy(xwsr[sd], out_sp.at[cidx[si]], add=True)

    def compute(sl):
        xb, db = xwsr[sl], dcol[sl]
        def edge(i, _2):
            a = xb[i, pl.ds(TH, 16)] + db[i]
            lr = jnp.maximum(a, a * 0.2)
            pe = jnp.exp(lr)
            xb[i, pl.ds(TH, 16)] = pe
            for h in range(HEADS):
                ph = pe[h]
                blk = xb[i, pl.ds(h * 16, 16)]
                xb[i, pl.ds(h * 16, 16)] = blk * ph
            return 0
        lax.fori_loop(0, C1, edge, 0)

    def stage(c, r, fi, fg):
        if fi:
            fire_idx(c + 2, (r + 2) % 4)
        if fg:
            wait_idx(c + 1, (r + 1) % 4)
            fire_gather((r + 1) % 2, (r + 1) % 4)
        wait_gather(r % 2, r % 4)
        compute(r % 2)
        fire_scatter(r % 2, r % 4)

    fire_idx(0, 0)
    fire_idx(1, 1)
    wait_idx(0, 0)
    fire_gather(0, 0)
    stage(0, 0, True, True)
    stage(1, 1, True, True)
    def steady(it, _):
        for j in range(4):
            stage(2 + it * 4 + j, (2 + j) % 4, True, True)
        return 0
    lax.fori_loop(0, (NCH1 - 4) // 4, steady, 0)
    stage(NCH1 - 2, (NCH1 - 2) % 4, False, True)
    stage(NCH1 - 1, (NCH1 - 1) % 4, False, False)
    plsc.subcore_barrier()

    pltpu.sync_copy(out_sp.at[pl.ds(zrow, RPT)], outp.at[cid, pl.ds(zrow, RPT)])


def _gat_edge_pass(rowsl, colsl, xwsrc, dst2):
    f = pl.kernel(
        _gat_edge_body,
        out_type=jax.ShapeDtypeStruct((NC, NP, TW), jnp.float32),
        mesh=_MESH,
        scratch_types=(
            *([pltpu.VMEM((C1,), jnp.int32)] * 8),
            *([pltpu.VMEM((C1, TW), jnp.float32)] * 2),
            *([pltpu.VMEM((C1, 16), jnp.float32)] * 2),
            pltpu.VMEM_SHARED((NP, TW), jnp.float32),
            *([pltpu.SemaphoreType.DMA] * 6),
        ),
        compiler_params=_SC_PARAMS,
    )
    return f(rowsl, colsl, xwsrc, dst2)


def _scorer_body(rowp, colp, tpad, abuf_h, bbuf_h, hb_h, w2c_h, sb2v_h,
                 lgw_out, aggp,
                 ridx0, ridx1, ridx2, ridx3,
                 cidx0, cidx1, cidx2, cidx3,
                 tbuf0, tbuf1, tbuf2, tbuf3,
                 arow0, arow1,
                 bcol0, bcol1,
                 hcol0, hcol1,
                 lgw0, lgw1,
                 sidx0, sidx1,
                 w2f, sb2b, agg_sp,
                 isem0, isem1, isem2, isem3,
                 gsem0, gsem1):
    cid = lax.axis_index("c")
    sid = lax.axis_index("s")
    w = cid * NS + sid
    ridx = [ridx0, ridx1, ridx2, ridx3]
    cidx = [cidx0, cidx1, cidx2, cidx3]
    tbuf = [tbuf0, tbuf1, tbuf2, tbuf3]
    arow = [arow0, arow1]
    bcol = [bcol0, bcol1]
    hcol = [hcol0, hcol1]
    lgw = [lgw0, lgw1]
    sidx = [sidx0, sidx1]
    isem = [isem0, isem1, isem2, isem3]
    gsem = [gsem0, gsem1]

    zrow = sid * RPT
    def _z(i, _):
        for k in range(8):
            hcol0[i, pl.ds(k * 16, 16)] = jnp.zeros((16,), jnp.float32)
        return 0
    lax.fori_loop(0, C2, _z, 0)
    nfull = RPT // C2
    def _zs(i, _):
        pltpu.sync_copy(hcol0, agg_sp.at[pl.ds(zrow + i * C2, C2)])
        return 0
    lax.fori_loop(0, nfull, _zs, 0)
    rem = RPT - nfull * C2
    if rem:
        pltpu.sync_copy(hcol0.at[pl.ds(0, rem)],
                        agg_sp.at[pl.ds(zrow + nfull * C2, rem)])
    pltpu.sync_copy(w2c_h, w2f)
    pltpu.sync_copy(sb2v_h, sb2b)
    plsc.subcore_barrier()

    def fire_idx(c, sl):
        gb = (w * NCH2 + c) * C2
        pltpu.async_copy(rowp.at[pl.ds(gb, C2)], ridx[sl], isem[sl])
        pltpu.async_copy(colp.at[pl.ds(gb, C2)], cidx[sl], isem[sl])
        pltpu.async_copy(tpad.at[pl.ds(gb, C2)], tbuf[sl], isem[sl])

    def wait_idx(c, sl):
        gb = (w * NCH2 + c) * C2
        pltpu.make_async_copy(rowp.at[pl.ds(gb, C2)], ridx[sl], isem[sl]).wait()
        pltpu.make_async_copy(colp.at[pl.ds(gb, C2)], cidx[sl], isem[sl]).wait()
        pltpu.make_async_copy(tpad.at[pl.ds(gb, C2)], tbuf[sl], isem[sl]).wait()

    def fire_gather(sd, si):
        pltpu.async_copy(abuf_h.at[ridx[si]], arow[sd], gsem[sd])
        pltpu.async_copy(bbuf_h.at[cidx[si]], bcol[sd], gsem[sd])
        pltpu.async_copy(hb_h.at[cidx[si]], hcol[sd], gsem[sd])

    def wait_gather(sd, si):
        pltpu.make_async_copy(abuf_h.at[ridx[si]], arow[sd], gsem[sd]).wait()
        pltpu.make_async_copy(bbuf_h.at[cidx[si]], bcol[sd], gsem[sd]).wait()
        pltpu.make_async_copy(hb_h.at[cidx[si]], hcol[sd], gsem[sd]).wait()

    def fire_scatter(c, sd):
        base = (w * NCH2 + c) * (2 * C2)
        pltpu.sync_copy(lgw[sd], lgw_out.at[pl.ds(base, 2 * C2)])
        pltpu.sync_copy(hcol[sd], agg_sp.at[sidx[sd]], add=True)

    iota = lax.iota(jnp.int32, 16)

    def compute(sd, si):
        ar, bc = arow[sd], bcol[sd]
        lb, sx = lgw[sd], sidx[sd]
        tb, rx = tbuf[si], ridx[si]
        sb2 = sb2b[...]
        def group(j, _2):
            base16 = j * 16
            eidx = base16 + iota
            def kbody(k, acc):
                kv = jnp.full((16,), k, jnp.int32)
                av = plsc.load_gather(ar, [eidx, kv])
                bv = plsc.load_gather(bc, [eidx, kv])
                w2v = plsc.load_gather(w2f, [kv])
                return acc + jnp.maximum(av + bv, 0.0) * w2v
            acc = lax.fori_loop(0, 64, kbody, sb2, unroll=8)
            tv = tb[pl.ds(base16, 16)]
            rv = rx[pl.ds(base16, 16)]
            m = acc > tv
            lb[pl.ds(base16, 16)] = acc
            lb[pl.ds(C2 + base16, 16)] = jnp.where(m, 1.0, 0.0)
            sx[pl.ds(base16, 16)] = jnp.where(m, rv, ND)
            return 0
        lax.fori_loop(0, C2 // 16, group, 0)

    def stage(c, r, fi, fg):
        if fi:
            fire_idx(c + 2, (r + 2) % 4)
        if fg:
            wait_idx(c + 1, (r + 1) % 4)
            fire_gather((r + 1) % 2, (r + 1) % 4)
        wait_gather(r % 2, r % 4)
        compute(r % 2, r % 4)
        fire_scatter(c, r % 2)

    fire_idx(0, 0)
    fire_idx(1, 1)
    wait_idx(0, 0)
    fire_gather(0, 0)
    stage(0, 0, True, True)
    stage(1, 1, True, True)
    def steady(it, _):
        for j in range(4):
            stage(2 + it * 4 + j, (2 + j) % 4, True, True)
        return 0
    lax.fori_loop(0, (NCH2 - 4) // 4, steady, 0)
    stage(NCH2 - 2, (NCH2 - 2) % 4, False, True)
    stage(NCH2 - 1, (NCH2 - 1) % 4, False, False)
    plsc.subcore_barrier()

    pltpu.sync_copy(agg_sp.at[pl.ds(zrow, RPT)], aggp.at[cid, pl.ds(zrow, RPT)])


def _scorer_pass(rowp, colp, tpad, abuf, bbuf, hb, w2c, sb2v):
    f = pl.kernel(
        _scorer_body,
        out_type=(
            jax.ShapeDtypeStruct((NW * NCH2 * 2 * C2,), jnp.float32),
            jax.ShapeDtypeStruct((NC, NP, TH), jnp.float32),
        ),
        mesh=_MESH,
        scratch_types=(
            *([pltpu.VMEM((C2,), jnp.int32)] * 8),
            *([pltpu.VMEM((C2,), jnp.float32)] * 4),
            *([pltpu.VMEM((C2, 64), jnp.float32)] * 4),
            *([pltpu.VMEM((C2, TH), jnp.float32)] * 2),
            *([pltpu.VMEM((2 * C2,), jnp.float32)] * 2),
            *([pltpu.VMEM((C2,), jnp.int32)] * 2),
            pltpu.VMEM((64,), jnp.float32),
            pltpu.VMEM((16,), jnp.float32),
            pltpu.VMEM_SHARED((NP, TH), jnp.float32),
            *([pltpu.SemaphoreType.DMA] * 6),
        ),
        compiler_params=_SC_PARAMS,
    )
    return f(rowp, colp, tpad, abuf, bbuf, hb, w2c, sb2v)


def _kernel_impl(x, edge_index, params, consts):
    p = params
    row = edge_index[0]
    col = edge_index[1]
    (a1cat, a2cat, em, s1, t1, s2, t2, cs, ct, w2p, b2p, tfix) = consts

    sl = jnp.arange(N, dtype=jnp.int32)
    padE1 = jnp.full((EP1 - ESL,), ND, jnp.int32)
    rowsl = jnp.concatenate([row, sl, padE1])
    colsl = jnp.concatenate([col, sl, padE1])
    padE2 = jnp.full((EP2 - E,), ND, jnp.int32)
    rowp = jnp.concatenate([row, padE2])
    colp = jnp.concatenate([col, padE2])
    tpad = jnp.concatenate([tfix, jnp.full((EP2 - E,), 1e30, jnp.float32)])

    xpad = jnp.zeros((NP, D_IN), jnp.float32).at[:N].set(x)

    rb2 = p["res_b"].reshape(1, TH)
    g1b = p["g1_b"].reshape(1, TH)
    g2b = p["g2_b"].reshape(1, TH)

    xws1, d2a = _tc_call(
        _k1_body,
        [_row_spec(_RB, D_IN), _full_spec((D_IN, TH)), _full_spec((1, TH)),
         _full_spec((TH, TH)), _full_spec((TH, 32))],
        [_row_spec(_RB, TW), _row_spec(_RB, 16)],
        [jax.ShapeDtypeStruct((NP, TW), jnp.float32),
         jax.ShapeDtypeStruct((NP, 16), jnp.float32)],
        [xpad, p["res_W"], rb2, p["g1_W"], a1cat],
    )

    outp1 = _gat_edge_pass(rowsl, colsl, xws1, d2a)

    xws2, d2b = _tc_call(
        _k2_body,
        [pl.BlockSpec((NC, _RB, TW), lambda i: (0, i, 0)),
         _full_spec((16, TH)), _full_spec((1, TH)), _full_spec((1, TH)),
         _full_spec((1, TH)), _full_spec((TH, TH)), _full_spec((TH, 32))],
        [_row_spec(_RB, TW), _row_spec(_RB, 16)],
        [jax.ShapeDtypeStruct((NP, TW), jnp.float32),
         jax.ShapeDtypeStruct((NP, 16), jnp.float32)],
        [outp1, em, g1b, s1.reshape(1, TH), t1.reshape(1, TH),
         p["g2_W"], a2cat],
    )

    outp2 = _gat_edge_pass(rowsl, colsl, xws2, d2b)

    hb, abuf, bbuf = _tc_call(
        _k3_body,
        [pl.BlockSpec((NC, _RB, TW), lambda i: (0, i, 0)),
         _full_spec((16, TH)), _full_spec((1, TH)), _full_spec((1, TH)),
         _full_spec((1, TH)), _full_spec((TH, 64)), _full_spec((TH, 64)),
         _full_spec((1, 64))],
        [_row_spec(_RB, TH), _row_spec(_RB, 64), _row_spec(_RB, 64)],
        [jax.ShapeDtypeStruct((NP, TH), jnp.float32),
         jax.ShapeDtypeStruct((NP, 64), jnp.float32),
         jax.ShapeDtypeStruct((NP, 64), jnp.float32)],
        [outp2, em, g2b, s2.reshape(1, TH), t2.reshape(1, TH),
         p["s_W1"][:TH], p["s_W1"][TH:], p["s_b1"].reshape(1, 64)],
    )

    w2c = p["s_W2"][:, 0]
    sb2v = jnp.full((16,), p["s_b2"][0], jnp.float32)
    lgw, aggp = _scorer_pass(rowp, colp, tpad, abuf, bbuf, hb, w2c, sb2v)

    (out,) = _tc_call(
        _k4_body,
        [_row_spec(_RB, TH),
         pl.BlockSpec((NC, _RB, TH), lambda i: (0, i, 0)),
         _full_spec((TH, 64)), _full_spec((1, 64)), _full_spec((1, 64)),
         _full_spec((1, 64)), _full_spec((64, TH)), _full_spec((1, TH))],
        [_row_spec(_RB, TH)],
        [jax.ShapeDtypeStruct((NP, TH), jnp.float32)],
        [hb, aggp, p["c_W1"], p["c_b1"].reshape(1, 64), cs.reshape(1, 64),
         ct.reshape(1, 64), w2p, b2p.reshape(1, TH)],
    )

    lgw2 = lgw.reshape(NW * NCH2, 2, C2)
    logits = lgw2[:, 0, :].reshape(EP2)[:E]
    weights = lgw2[:, 1, :].reshape(EP2)[:E]
    return out[:N, :OUT], weights, logits


def _make_consts(params):
    p = params

    def acat(a_s, a_d):
        eye = jnp.eye(HEADS, dtype=jnp.float32)
        ms = (a_s[:, :, None] * eye[:, None, :]).reshape(TH, HEADS)
        md = (a_d[:, :, None] * eye[:, None, :]).reshape(TH, HEADS)
        return jnp.concatenate([ms, ms, md, md], axis=1)

    a1cat = acat(p["g1_as"], p["g1_ad"])
    a2cat = acat(p["g2_as"], p["g2_ad"])
    em = jnp.concatenate(
        [jnp.kron(jnp.eye(HEADS, dtype=jnp.float32), jnp.ones((1, 16), jnp.float32)),
         jnp.zeros((8, TH), jnp.float32)], axis=0)

    def bnst(g, b, m, v):
        s = g / jnp.sqrt(v + 1e-5)
        return s, b - m * s

    s1, t1 = bnst(p["bn1_g"], p["bn1_b"], p["bn1_m"], p["bn1_v"])
    s2, t2 = bnst(p["bn2_g"], p["bn2_b"], p["bn2_m"], p["bn2_v"])
    cs, ct = bnst(p["cbn_g"], p["cbn_b"], p["cbn_m"], p["cbn_v"])

    w2p = jnp.zeros((64, TH), jnp.float32).at[:, :OUT].set(p["c_W2"])
    b2p = jnp.full((TH,), -1e30, jnp.float32).at[:OUT].set(p["c_b2"])

    u = jax.random.uniform(jax.random.key(42), (E, 2),
                           minval=1e-6, maxval=1.0 - 1e-6)
    g = -jnp.log(-jnp.log(u))
    tfix = g[:, 0] - g[:, 1]

    return (a1cat, a2cat, em, s1, t1, s2, t2, cs, ct, w2p, b2p, tfix)


@jax.jit
def kernel(x, edge_index, params):
    consts = _make_consts(params)
    return _kernel_impl(x, edge_index.astype(jnp.int32), params, consts)


# speedup vs baseline: 1.4073x; 1.0021x over previous
import jax
import jax.numpy as jnp
from jax import lax
from jax.experimental import pallas as pl
from jax.experimental.pallas import tpu as pltpu
from jax.experimental.pallas import tpu_sc as plsc

N = 10000
E = 320000
D_IN = 128
HEADS = 8
TH = 128
OUT = 40

NP = 10240
ND = N
NC = 2
NS = 16
NW = NC * NS
RPT = NP // NS
TW = TH + 16

ESL = E + N

C1 = 112
NCH1 = 96
EPW1 = NCH1 * C1
EP1 = EPW1 * NW

C2 = 80
NCH2 = 132
EPW2 = NCH2 * C2
EP2 = EPW2 * NW

_HI = jax.lax.Precision.HIGHEST


def _dot(a, b):
    return jax.lax.dot_general(a, b, (((1,), (0,)), ((), ())),
                               precision=_HI, preferred_element_type=jnp.float32)


def _k1_body(x_ref, rw_ref, rb_ref, gw_ref, ac_ref,
             xws_ref, d2_ref):
    xp = _dot(x_ref[...], rw_ref[...]) + rb_ref[...]
    xw = _dot(xp, gw_ref[...])
    sd = _dot(xw, ac_ref[...])
    xws_ref[...] = jnp.concatenate([xw, sd[:, :16]], axis=1)
    d2_ref[...] = sd[:, 16:]


def _k2_body(op_ref, em_ref, gb_ref, s_ref, t_ref, gw_ref, ac_ref,
             xws_ref, d2_ref):
    un = op_ref[0] + op_ref[1]
    dex = _dot(un[:, TH:], em_ref[...])
    g = un[:, :TH] / (dex + 1e-16) + gb_ref[...]
    g = g * s_ref[...] + t_ref[...]
    h = jnp.where(g > 0, g, jnp.exp(g) - 1.0)
    xw = _dot(h, gw_ref[...])
    sd = _dot(xw, ac_ref[...])
    xws_ref[...] = jnp.concatenate([xw, sd[:, :16]], axis=1)
    d2_ref[...] = sd[:, 16:]


def _k3_body(op_ref, em_ref, gb_ref, s_ref, t_ref, wa_ref, wb_ref, b1_ref,
             hb_ref, a_ref, b_ref):
    un = op_ref[0] + op_ref[1]
    dex = _dot(un[:, TH:], em_ref[...])
    g = un[:, :TH] / (dex + 1e-16) + gb_ref[...]
    g = g * s_ref[...] + t_ref[...]
    h = jnp.where(g > 0, g, jnp.exp(g) - 1.0)
    hb_ref[...] = h
    a_ref[...] = _dot(h, wa_ref[...])
    b_ref[...] = _dot(h, wb_ref[...]) + b1_ref[...]


def _k4_body(hb_ref, ag_ref, w1_ref, b1_ref, s_ref, t_ref, w2_ref, b2_ref,
             out_ref):
    hs = hb_ref[...] + ag_ref[0] + ag_ref[1]
    c1 = _dot(hs, w1_ref[...]) + b1_ref[...]
    c1 = c1 * s_ref[...] + t_ref[...]
    c1 = jnp.maximum(c1, 0.0)
    lg = _dot(c1, w2_ref[...]) + b2_ref[...]
    m = jnp.max(lg, axis=1, keepdims=True)
    lse = m + jnp.log(jnp.sum(jnp.exp(lg - m), axis=1, keepdims=True))
    out_ref[...] = lg - lse


def _row_spec(rb, cols):
    return pl.BlockSpec((rb, cols), lambda i: (i, 0))


def _full_spec(shape):
    nd = len(shape)
    return pl.BlockSpec(shape, lambda i: (0,) * nd)


_RB = 1024
_GRID = NP // _RB


def _tc_call(body, in_specs, out_specs, out_shapes, args):
    return pl.pallas_call(
        body,
        grid=(_GRID,),
        in_specs=in_specs,
        out_specs=out_specs,
        out_shape=out_shapes,
    )(*args)


_MESH = plsc.VectorSubcoreMesh(core_axis_name="c", subcore_axis_name="s")
_SC_PARAMS = pltpu.CompilerParams(use_tc_tiling_on_sc=False,
                                  needs_layout_passes=False)


def _gat_edge_body(rowsl, colsl, xwsrc, dst2, outp,
                   ridx0, ridx1, ridx2, ridx3,
                   cidx0, cidx1, cidx2, cidx3,
                   xwsr0, xwsr1,
                   dcol0, dcol1,
                   out_sp,
                   isem0, isem1, isem2, isem3,
                   gsem0, gsem1):
    cid = lax.axis_index("c")
    sid = lax.axis_index("s")
    w = cid * NS + sid
    ridx = [ridx0, ridx1, ridx2, ridx3]
    cidx = [cidx0, cidx1, cidx2, cidx3]
    xwsr = [xwsr0, xwsr1]
    dcol = [dcol0, dcol1]
    isem = [isem0, isem1, isem2, isem3]
    gsem = [gsem0, gsem1]

    zrow = sid * RPT
    def _z(i, _):
        for k in range(TW // 16):
            xwsr0[i, pl.ds(k * 16, 16)] = jnp.zeros((16,), jnp.float32)
        return 0
    lax.fori_loop(0, C1, _z, 0)
    nfull = RPT // C1
    def _zs(i, _):
        pltpu.sync_copy(xwsr0, out_sp.at[pl.ds(zrow + i * C1, C1)])
        return 0
    lax.fori_loop(0, nfull, _zs, 0)
    rem = RPT - nfull * C1
    if rem:
        pltpu.sync_copy(xwsr0.at[pl.ds(0, rem)],
                        out_sp.at[pl.ds(zrow + nfull * C1, rem)])
    plsc.subcore_barrier()

    def fire_idx(c, sl):
        gb = (w * NCH1 + c) * C1
        pltpu.async_copy(rowsl.at[pl.ds(gb, C1)], ridx[sl], isem[sl])
        pltpu.async_copy(colsl.at[pl.ds(gb, C1)], cidx[sl], isem[sl])

    def wait_idx(c, sl):
        gb = (w * NCH1 + c) * C1
        pltpu.make_async_copy(rowsl.at[pl.ds(gb, C1)], ridx[sl], isem[sl]).wait()
        pltpu.make_async_copy(colsl.at[pl.ds(gb, C1)], cidx[sl], isem[sl]).wait()

    def fire_gather(sd, si):
        pltpu.async_copy(xwsrc.at[ridx[si]], xwsr[sd], gsem[sd])
        pltpu.async_copy(dst2.at[cidx[si]], dcol[sd], gsem[sd])

    def wait_gather(sd, si):
        pltpu.make_async_copy(xwsrc.at[ridx[si]], xwsr[sd], gsem[sd]).wait()
        pltpu.make_async_copy(dst2.at[cidx[si]], dcol[sd], gsem[sd]).wait()

    def fire_scatter(sd, si):
        pltpu.sync_copy(xwsr[sd], out_sp.at[cidx[si]], add=True)

    def compute(sl):
        xb, db = xwsr[sl], dcol[sl]
        def edge(i, _2):
            a = xb[i, pl.ds(TH, 16)] + db[i]
            lr = jnp.maximum(a, a * 0.2)
            pe = jnp.exp(lr)
            xb[i, pl.ds(TH, 16)] = pe
            for h in range(HEADS):
                ph = pe[h]
                blk = xb[i, pl.ds(h * 16, 16)]
                xb[i, pl.ds(h * 16, 16)] = blk * ph
            return 0
        lax.fori_loop(0, C1, edge, 0)

    def stage(c, r, fi, fg):
        if fi:
            fire_idx(c + 2, (r + 2) % 4)
        if fg:
            wait_idx(c + 1, (r + 1) % 4)
            fire_gather((r + 1) % 2, (r + 1) % 4)
        wait_gather(r % 2, r % 4)
        compute(r % 2)
        fire_scatter(r % 2, r % 4)

    fire_idx(0, 0)
    fire_idx(1, 1)
    wait_idx(0, 0)
    fire_gather(0, 0)
    stage(0, 0, True, True)
    stage(1, 1, True, True)
    def steady(it, _):
        for j in range(4):
            stage(2 + it * 4 + j, (2 + j) % 4, True, True)
        return 0
    lax.fori_loop(0, (NCH1 - 4) // 4, steady, 0)
    stage(NCH1 - 2, (NCH1 - 2) % 4, False, True)
    stage(NCH1 - 1, (NCH1 - 1) % 4, False, False)
    plsc.subcore_barrier()

    pltpu.sync_copy(out_sp.at[pl.ds(zrow, RPT)], outp.at[cid, pl.ds(zrow, RPT)])


def _gat_edge_pass(rowsl, colsl, xwsrc, dst2):
    f = pl.kernel(
        _gat_edge_body,
        out_type=jax.ShapeDtypeStruct((NC, NP, TW), jnp.float32),
        mesh=_MESH,
        scratch_types=(
            *([pltpu.VMEM((C1,), jnp.int32)] * 8),
            *([pltpu.VMEM((C1, TW), jnp.float32)] * 2),
            *([pltpu.VMEM((C1, 16), jnp.float32)] * 2),
            pltpu.VMEM_SHARED((NP, TW), jnp.float32),
            *([pltpu.SemaphoreType.DMA] * 6),
        ),
        compiler_params=_SC_PARAMS,
    )
    return f(rowsl, colsl, xwsrc, dst2)


def _scorer_body(rowp, colp, tpad, abuf_h, bbuf_h, hb_h, w2c_h, sb2v_h,
                 lgw_out, aggp,
                 ridx0, ridx1, ridx2, ridx3,
                 cidx0, cidx1, cidx2, cidx3,
                 tbuf0, tbuf1, tbuf2, tbuf3,
                 arow0, arow1,
                 bcol0, bcol1,
                 hcol0, hcol1,
                 lgw0, lgw1,
                 sidx0, sidx1,
                 w2f, sb2b, agg_sp,
                 isem0, isem1, isem2, isem3,
                 gsem0, gsem1, lsem0, lsem1):
    cid = lax.axis_index("c")
    sid = lax.axis_index("s")
    w = cid * NS + sid
    ridx = [ridx0, ridx1, ridx2, ridx3]
    cidx = [cidx0, cidx1, cidx2, cidx3]
    tbuf = [tbuf0, tbuf1, tbuf2, tbuf3]
    arow = [arow0, arow1]
    bcol = [bcol0, bcol1]
    hcol = [hcol0, hcol1]
    lgw = [lgw0, lgw1]
    sidx = [sidx0, sidx1]
    isem = [isem0, isem1, isem2, isem3]
    gsem = [gsem0, gsem1]
    lsem = [lsem0, lsem1]

    zrow = sid * RPT
    def _z(i, _):
        for k in range(8):
            hcol0[i, pl.ds(k * 16, 16)] = jnp.zeros((16,), jnp.float32)
        return 0
    lax.fori_loop(0, C2, _z, 0)
    nfull = RPT // C2
    def _zs(i, _):
        pltpu.sync_copy(hcol0, agg_sp.at[pl.ds(zrow + i * C2, C2)])
        return 0
    lax.fori_loop(0, nfull, _zs, 0)
    rem = RPT - nfull * C2
    if rem:
        pltpu.sync_copy(hcol0.at[pl.ds(0, rem)],
                        agg_sp.at[pl.ds(zrow + nfull * C2, rem)])
    pltpu.sync_copy(w2c_h, w2f)
    pltpu.sync_copy(sb2v_h, sb2b)
    plsc.subcore_barrier()

    def fire_idx(c, sl):
        gb = (w * NCH2 + c) * C2
        pltpu.async_copy(rowp.at[pl.ds(gb, C2)], ridx[sl], isem[sl])
        pltpu.async_copy(colp.at[pl.ds(gb, C2)], cidx[sl], isem[sl])
        pltpu.async_copy(tpad.at[pl.ds(gb, C2)], tbuf[sl], isem[sl])

    def wait_idx(c, sl):
        gb = (w * NCH2 + c) * C2
        pltpu.make_async_copy(rowp.at[pl.ds(gb, C2)], ridx[sl], isem[sl]).wait()
        pltpu.make_async_copy(colp.at[pl.ds(gb, C2)], cidx[sl], isem[sl]).wait()
        pltpu.make_async_copy(tpad.at[pl.ds(gb, C2)], tbuf[sl], isem[sl]).wait()

    def fire_gather(sd, si):
        pltpu.async_copy(abuf_h.at[ridx[si]], arow[sd], gsem[sd])
        pltpu.async_copy(bbuf_h.at[cidx[si]], bcol[sd], gsem[sd])
        pltpu.async_copy(hb_h.at[cidx[si]], hcol[sd], gsem[sd])

    def wait_gather(sd, si):
        pltpu.make_async_copy(abuf_h.at[ridx[si]], arow[sd], gsem[sd]).wait()
        pltpu.make_async_copy(bbuf_h.at[cidx[si]], bcol[sd], gsem[sd]).wait()
        pltpu.make_async_copy(hb_h.at[cidx[si]], hcol[sd], gsem[sd]).wait()

    def fire_scatter(c, sd):
        base = (w * NCH2 + c) * (2 * C2)
        pltpu.async_copy(lgw[sd], lgw_out.at[pl.ds(base, 2 * C2)], lsem[sd])
        pltpu.sync_copy(hcol[sd], agg_sp.at[sidx[sd]], add=True)

    def wait_lgw(sd):
        pltpu.make_async_copy(lgw_out.at[pl.ds(0, 2 * C2)], lgw[sd], lsem[sd]).wait()

    iota = lax.iota(jnp.int32, 16)

    def compute(sd, si):
        ar, bc = arow[sd], bcol[sd]
        lb, sx = lgw[sd], sidx[sd]
        tb, rx = tbuf[si], ridx[si]
        sb2 = sb2b[...]
        def group(j, _2):
            base16 = j * 16
            eidx = base16 + iota
            def kbody(k, acc):
                kv = jnp.full((16,), k, jnp.int32)
                av = plsc.load_gather(ar, [eidx, kv])
                bv = plsc.load_gather(bc, [eidx, kv])
                w2v = plsc.load_gather(w2f, [kv])
                return acc + jnp.maximum(av + bv, 0.0) * w2v
            acc = lax.fori_loop(0, 64, kbody, sb2, unroll=8)
            tv = tb[pl.ds(base16, 16)]
            rv = rx[pl.ds(base16, 16)]
            m = acc > tv
            lb[pl.ds(base16, 16)] = acc
            lb[pl.ds(C2 + base16, 16)] = jnp.where(m, 1.0, 0.0)
            sx[pl.ds(base16, 16)] = jnp.where(m, rv, ND)
            return 0
        lax.fori_loop(0, C2 // 16, group, 0)

    def stage(c, r, wl, fi, fg):
        if wl:
            wait_lgw(r % 2)
        if fi:
            fire_idx(c + 2, (r + 2) % 4)
        if fg:
            wait_idx(c + 1, (r + 1) % 4)
            fire_gather((r + 1) % 2, (r + 1) % 4)
        wait_gather(r % 2, r % 4)
        compute(r % 2, r % 4)
        fire_scatter(c, r % 2)

    fire_idx(0, 0)
    fire_idx(1, 1)
    wait_idx(0, 0)
    fire_gather(0, 0)
    stage(0, 0, False, True, True)
    stage(1, 1, False, True, True)
    def steady(it, _):
        for j in range(4):
            stage(2 + it * 4 + j, (2 + j) % 4, True, True, True)
        return 0
    lax.fori_loop(0, (NCH2 - 4) // 4, steady, 0)
    stage(NCH2 - 2, (NCH2 - 2) % 4, True, False, True)
    stage(NCH2 - 1, (NCH2 - 1) % 4, True, False, False)
    wait_lgw((NCH2 - 2) % 2)
    wait_lgw((NCH2 - 1) % 2)
    plsc.subcore_barrier()

    pltpu.sync_copy(agg_sp.at[pl.ds(zrow, RPT)], aggp.at[cid, pl.ds(zrow, RPT)])


def _scorer_pass(rowp, colp, tpad, abuf, bbuf, hb, w2c, sb2v):
    f = pl.kernel(
        _scorer_body,
        out_type=(
            jax.ShapeDtypeStruct((NW * NCH2 * 2 * C2,), jnp.float32),
            jax.ShapeDtypeStruct((NC, NP, TH), jnp.float32),
        ),
        mesh=_MESH,
        scratch_types=(
            *([pltpu.VMEM((C2,), jnp.int32)] * 8),
            *([pltpu.VMEM((C2,), jnp.float32)] * 4),
            *([pltpu.VMEM((C2, 64), jnp.float32)] * 4),
            *([pltpu.VMEM((C2, TH), jnp.float32)] * 2),
            *([pltpu.VMEM((2 * C2,), jnp.float32)] * 2),
            *([pltpu.VMEM((C2,), jnp.int32)] * 2),
            pltpu.VMEM((64,), jnp.float32),
            pltpu.VMEM((16,), jnp.float32),
            pltpu.VMEM_SHARED((NP, TH), jnp.float32),
            *([pltpu.SemaphoreType.DMA] * 8),
        ),
        compiler_params=_SC_PARAMS,
    )
    return f(rowp, colp, tpad, abuf, bbuf, hb, w2c, sb2v)


def _kernel_impl(x, edge_index, params, consts):
    p = params
    row = edge_index[0]
    col = edge_index[1]
    (a1cat, a2cat, em, s1, t1, s2, t2, cs, ct, w2p, b2p, tfix) = consts

    sl = jnp.arange(N, dtype=jnp.int32)
    padE1 = jnp.full((EP1 - ESL,), ND, jnp.int32)
    rowsl = jnp.concatenate([row, sl, padE1])
    colsl = jnp.concatenate([col, sl, padE1])
    padE2 = jnp.full((EP2 - E,), ND, jnp.int32)
    rowp = jnp.concatenate([row, padE2])
    colp = jnp.concatenate([col, padE2])
    tpad = jnp.concatenate([tfix, jnp.full((EP2 - E,), 1e30, jnp.float32)])

    xpad = jnp.zeros((NP, D_IN), jnp.float32).at[:N].set(x)

    rb2 = p["res_b"].reshape(1, TH)
    g1b = p["g1_b"].reshape(1, TH)
    g2b = p["g2_b"].reshape(1, TH)

    xws1, d2a = _tc_call(
        _k1_body,
        [_row_spec(_RB, D_IN), _full_spec((D_IN, TH)), _full_spec((1, TH)),
         _full_spec((TH, TH)), _full_spec((TH, 32))],
        [_row_spec(_RB, TW), _row_spec(_RB, 16)],
        [jax.ShapeDtypeStruct((NP, TW), jnp.float32),
         jax.ShapeDtypeStruct((NP, 16), jnp.float32)],
        [xpad, p["res_W"], rb2, p["g1_W"], a1cat],
    )

    outp1 = _gat_edge_pass(rowsl, colsl, xws1, d2a)

    xws2, d2b = _tc_call(
        _k2_body,
        [pl.BlockSpec((NC, _RB, TW), lambda i: (0, i, 0)),
         _full_spec((16, TH)), _full_spec((1, TH)), _full_spec((1, TH)),
         _full_spec((1, TH)), _full_spec((TH, TH)), _full_spec((TH, 32))],
        [_row_spec(_RB, TW), _row_spec(_RB, 16)],
        [jax.ShapeDtypeStruct((NP, TW), jnp.float32),
         jax.ShapeDtypeStruct((NP, 16), jnp.float32)],
        [outp1, em, g1b, s1.reshape(1, TH), t1.reshape(1, TH),
         p["g2_W"], a2cat],
    )

    outp2 = _gat_edge_pass(rowsl, colsl, xws2, d2b)

    hb, abuf, bbuf = _tc_call(
        _k3_body,
        [pl.BlockSpec((NC, _RB, TW), lambda i: (0, i, 0)),
         _full_spec((16, TH)), _full_spec((1, TH)), _full_spec((1, TH)),
         _full_spec((1, TH)), _full_spec((TH, 64)), _full_spec((TH, 64)),
         _full_spec((1, 64))],
        [_row_spec(_RB, TH), _row_spec(_RB, 64), _row_spec(_RB, 64)],
        [jax.ShapeDtypeStruct((NP, TH), jnp.float32),
         jax.ShapeDtypeStruct((NP, 64), jnp.float32),
         jax.ShapeDtypeStruct((NP, 64), jnp.float32)],
        [outp2, em, g2b, s2.reshape(1, TH), t2.reshape(1, TH),
         p["s_W1"][:TH], p["s_W1"][TH:], p["s_b1"].reshape(1, 64)],
    )

    w2c = p["s_W2"][:, 0]
    sb2v = jnp.full((16,), p["s_b2"][0], jnp.float32)
    lgw, aggp = _scorer_pass(rowp, colp, tpad, abuf, bbuf, hb, w2c, sb2v)

    (out,) = _tc_call(
        _k4_body,
        [_row_spec(_RB, TH),
         pl.BlockSpec((NC, _RB, TH), lambda i: (0, i, 0)),
         _full_spec((TH, 64)), _full_spec((1, 64)), _full_spec((1, 64)),
         _full_spec((1, 64)), _full_spec((64, TH)), _full_spec((1, TH))],
        [_row_spec(_RB, TH)],
        [jax.ShapeDtypeStruct((NP, TH), jnp.float32)],
        [hb, aggp, p["c_W1"], p["c_b1"].reshape(1, 64), cs.reshape(1, 64),
         ct.reshape(1, 64), w2p, b2p.reshape(1, TH)],
    )

    lgw2 = lgw.reshape(NW * NCH2, 2, C2)
    logits = lgw2[:, 0, :].reshape(EP2)[:E]
    weights = lgw2[:, 1, :].reshape(EP2)[:E]
    return out[:N, :OUT], weights, logits


def _make_consts(params):
    p = params

    def acat(a_s, a_d):
        eye = jnp.eye(HEADS, dtype=jnp.float32)
        ms = (a_s[:, :, None] * eye[:, None, :]).reshape(TH, HEADS)
        md = (a_d[:, :, None] * eye[:, None, :]).reshape(TH, HEADS)
        return jnp.concatenate([ms, ms, md, md], axis=1)

    a1cat = acat(p["g1_as"], p["g1_ad"])
    a2cat = acat(p["g2_as"], p["g2_ad"])
    em = jnp.concatenate(
        [jnp.kron(jnp.eye(HEADS, dtype=jnp.float32), jnp.ones((1, 16), jnp.float32)),
         jnp.zeros((8, TH), jnp.float32)], axis=0)

    def bnst(g, b, m, v):
        s = g / jnp.sqrt(v + 1e-5)
        return s, b - m * s

    s1, t1 = bnst(p["bn1_g"], p["bn1_b"], p["bn1_m"], p["bn1_v"])
    s2, t2 = bnst(p["bn2_g"], p["bn2_b"], p["bn2_m"], p["bn2_v"])
    cs, ct = bnst(p["cbn_g"], p["cbn_b"], p["cbn_m"], p["cbn_v"])

    w2p = jnp.zeros((64, TH), jnp.float32).at[:, :OUT].set(p["c_W2"])
    b2p = jnp.full((TH,), -1e30, jnp.float32).at[:OUT].set(p["c_b2"])

    u = jax.random.uniform(jax.random.key(42), (E, 2),
                           minval=1e-6, maxval=1.0 - 1e-6)
    g = -jnp.log(-jnp.log(u))
    tfix = g[:, 0] - g[:, 1]

    return (a1cat, a2cat, em, s1, t1, s2, t2, cs, ct, w2p, b2p, tfix)


@jax.jit
def kernel(x, edge_index, params):
    consts = _make_consts(params)
    return _kernel_impl(x, edge_index.astype(jnp.int32), params, consts)


# speedup vs baseline: 2.0441x; 1.4525x over previous
import jax
import jax.numpy as jnp
from jax import lax
from jax.experimental import pallas as pl
from jax.experimental.pallas import tpu as pltpu
from jax.experimental.pallas import tpu_sc as plsc

N = 10000
E = 320000
D_IN = 128
HEADS = 8
TH = 128
OUT = 40

NP = 10240
ND = N
NC = 2
NS = 16
NW = NC * NS
RPT = NP // NS
TW = TH + 16

ESL = E + N

C1 = 112
NCH1 = 93
EPW1 = NCH1 * C1
EP1 = EPW1 * NW

C2 = 80
NCH2 = 125
EPW2 = NCH2 * C2
EP2 = EPW2 * NW

_HI = jax.lax.Precision.HIGHEST


def _dot(a, b):
    return jax.lax.dot_general(a, b, (((1,), (0,)), ((), ())),
                               precision=_HI, preferred_element_type=jnp.float32)


def _k1_body(x_ref, rw_ref, rb_ref, gw_ref, ac_ref,
             xws_ref, d2_ref):
    xp = _dot(x_ref[...], rw_ref[...]) + rb_ref[...]
    xw = _dot(xp, gw_ref[...])
    sd = _dot(xw, ac_ref[...])
    xws_ref[...] = jnp.concatenate([xw, sd[:, :16]], axis=1)
    d2_ref[...] = sd[:, 16:]


def _k2_body(op_ref, em_ref, gb_ref, s_ref, t_ref, gw_ref, ac_ref,
             xws_ref, d2_ref):
    un = op_ref[0] + op_ref[1]
    dex = _dot(un[:, TH:], em_ref[...])
    g = un[:, :TH] / (dex + 1e-16) + gb_ref[...]
    g = g * s_ref[...] + t_ref[...]
    h = jnp.where(g > 0, g, jnp.exp(g) - 1.0)
    xw = _dot(h, gw_ref[...])
    sd = _dot(xw, ac_ref[...])
    xws_ref[...] = jnp.concatenate([xw, sd[:, :16]], axis=1)
    d2_ref[...] = sd[:, 16:]


def _k3_body(op_ref, em_ref, gb_ref, s_ref, t_ref, wa_ref, wb_ref, b1_ref,
             hb_ref, a_ref, b_ref):
    un = op_ref[0] + op_ref[1]
    dex = _dot(un[:, TH:], em_ref[...])
    g = un[:, :TH] / (dex + 1e-16) + gb_ref[...]
    g = g * s_ref[...] + t_ref[...]
    h = jnp.where(g > 0, g, jnp.exp(g) - 1.0)
    hb_ref[...] = h
    a_ref[...] = _dot(h, wa_ref[...])
    b_ref[...] = _dot(h, wb_ref[...]) + b1_ref[...]


def _k4_body(hb_ref, ag_ref, w1_ref, b1_ref, s_ref, t_ref, w2_ref, b2_ref,
             out_ref):
    hs = hb_ref[...] + ag_ref[0] + ag_ref[1]
    c1 = _dot(hs, w1_ref[...]) + b1_ref[...]
    c1 = c1 * s_ref[...] + t_ref[...]
    c1 = jnp.maximum(c1, 0.0)
    lg = _dot(c1, w2_ref[...]) + b2_ref[...]
    m = jnp.max(lg, axis=1, keepdims=True)
    lse = m + jnp.log(jnp.sum(jnp.exp(lg - m), axis=1, keepdims=True))
    out_ref[...] = lg - lse


def _row_spec(rb, cols):
    return pl.BlockSpec((rb, cols), lambda i: (i, 0))


def _full_spec(shape):
    nd = len(shape)
    return pl.BlockSpec(shape, lambda i: (0,) * nd)


_RB = 1024
_GRID = NP // _RB


def _tc_call(body, in_specs, out_specs, out_shapes, args):
    return pl.pallas_call(
        body,
        grid=(_GRID,),
        in_specs=in_specs,
        out_specs=out_specs,
        out_shape=out_shapes,
    )(*args)


_MESH = plsc.VectorSubcoreMesh(core_axis_name="c", subcore_axis_name="s")
_SC_PARAMS = pltpu.CompilerParams(use_tc_tiling_on_sc=False,
                                  needs_layout_passes=False)


def _gat_edge_body(rowsl, colsl, xwsrc, dst2, outp,
                   ridx0, ridx1, ridx2, ridx3,
                   cidx0, cidx1, cidx2, cidx3,
                   xwsr0, xwsr1,
                   dcol0, dcol1,
                   out_sp,
                   isem0, isem1, isem2, isem3,
                   gsem0, gsem1):
    cid = lax.axis_index("c")
    sid = lax.axis_index("s")
    w = cid * NS + sid
    ridx = [ridx0, ridx1, ridx2, ridx3]
    cidx = [cidx0, cidx1, cidx2, cidx3]
    xwsr = [xwsr0, xwsr1]
    dcol = [dcol0, dcol1]
    isem = [isem0, isem1, isem2, isem3]
    gsem = [gsem0, gsem1]

    zrow = sid * RPT
    def _z(i, _):
        for k in range(TW // 16):
            xwsr0[i, pl.ds(k * 16, 16)] = jnp.zeros((16,), jnp.float32)
        return 0
    lax.fori_loop(0, C1, _z, 0)
    nfull = RPT // C1
    def _zs(i, _):
        pltpu.sync_copy(xwsr0, out_sp.at[pl.ds(zrow + i * C1, C1)])
        return 0
    lax.fori_loop(0, nfull, _zs, 0)
    rem = RPT - nfull * C1
    if rem:
        pltpu.sync_copy(xwsr0.at[pl.ds(0, rem)],
                        out_sp.at[pl.ds(zrow + nfull * C1, rem)])
    plsc.subcore_barrier()

    def fire_idx(c, sl):
        gb = (w * NCH1 + c) * C1
        pltpu.async_copy(rowsl.at[pl.ds(gb, C1)], ridx[sl], isem[sl])
        pltpu.async_copy(colsl.at[pl.ds(gb, C1)], cidx[sl], isem[sl])

    def wait_idx(c, sl):
        gb = (w * NCH1 + c) * C1
        pltpu.make_async_copy(rowsl.at[pl.ds(gb, C1)], ridx[sl], isem[sl]).wait()
        pltpu.make_async_copy(colsl.at[pl.ds(gb, C1)], cidx[sl], isem[sl]).wait()

    def fire_gather(sd, si):
        pltpu.async_copy(xwsrc.at[ridx[si]], xwsr[sd], gsem[sd])
        pltpu.async_copy(dst2.at[cidx[si]], dcol[sd], gsem[sd])

    def wait_gather(sd, si):
        pltpu.make_async_copy(xwsrc.at[ridx[si]], xwsr[sd], gsem[sd]).wait()
        pltpu.make_async_copy(dst2.at[cidx[si]], dcol[sd], gsem[sd]).wait()

    def fire_scatter(sd, si):
        pltpu.sync_copy(xwsr[sd], out_sp.at[cidx[si]], add=True)

    def compute(sl):
        xb, db = xwsr[sl], dcol[sl]
        def edge(i, _2):
            a = xb[i, pl.ds(TH, 16)] + db[i]
            lr = jnp.maximum(a, a * 0.2)
            pe = jnp.exp(lr)
            xb[i, pl.ds(TH, 16)] = pe
            for h in range(HEADS):
                ph = pe[h]
                blk = xb[i, pl.ds(h * 16, 16)]
                xb[i, pl.ds(h * 16, 16)] = blk * ph
            return 0
        lax.fori_loop(0, C1, edge, 0)

    def stage(c, r, fi, fg):
        if fi:
            fire_idx(c + 2, (r + 2) % 4)
        if fg:
            wait_idx(c + 1, (r + 1) % 4)
            fire_gather((r + 1) % 2, (r + 1) % 4)
        wait_gather(r % 2, r % 4)
        compute(r % 2)
        fire_scatter(r % 2, r % 4)

    fire_idx(0, 0)
    fire_idx(1, 1)
    wait_idx(0, 0)
    fire_gather(0, 0)
    stage(0, 0, True, True)
    stage(1, 1, True, True)
    def steady(it, _):
        for j in range(4):
            stage(2 + it * 4 + j, (2 + j) % 4, True, True)
        return 0
    lax.fori_loop(0, (NCH1 - 5) // 4, steady, 0)
    stage(NCH1 - 3, (NCH1 - 3) % 4, True, True)
    stage(NCH1 - 2, (NCH1 - 2) % 4, False, True)
    stage(NCH1 - 1, (NCH1 - 1) % 4, False, False)
    plsc.subcore_barrier()

    pltpu.sync_copy(out_sp.at[pl.ds(zrow, RPT)], outp.at[cid, pl.ds(zrow, RPT)])


def _gat_edge_pass(rowsl, colsl, xwsrc, dst2):
    f = pl.kernel(
        _gat_edge_body,
        out_type=jax.ShapeDtypeStruct((NC, NP, TW), jnp.float32),
        mesh=_MESH,
        scratch_types=(
            *([pltpu.VMEM((C1,), jnp.int32)] * 8),
            *([pltpu.VMEM((C1, TW), jnp.float32)] * 2),
            *([pltpu.VMEM((C1, 16), jnp.float32)] * 2),
            pltpu.VMEM_SHARED((NP, TW), jnp.float32),
            *([pltpu.SemaphoreType.DMA] * 6),
        ),
        compiler_params=_SC_PARAMS,
    )
    return f(rowsl, colsl, xwsrc, dst2)


def _scorer_body(rowp, colp, tpad, abuf_h, bbuf_h, hb_h, w2c_h, sb2v_h,
                 lgw_out, aggp,
                 ridx0, ridx1, ridx2, ridx3,
                 cidx0, cidx1, cidx2, cidx3,
                 tbuf0, tbuf1, tbuf2, tbuf3,
                 arow0, arow1,
                 bcol0, bcol1,
                 hcol0, hcol1,
                 lgw0, lgw1,
                 sidx0, sidx1,
                 w2f, sb2b, agg_sp,
                 isem0, isem1, isem2, isem3,
                 gsem0, gsem1, lsem0, lsem1):
    cid = lax.axis_index("c")
    sid = lax.axis_index("s")
    w = cid * NS + sid
    ridx = [ridx0, ridx1, ridx2, ridx3]
    cidx = [cidx0, cidx1, cidx2, cidx3]
    tbuf = [tbuf0, tbuf1, tbuf2, tbuf3]
    arow = [arow0, arow1]
    bcol = [bcol0, bcol1]
    hcol = [hcol0, hcol1]
    lgw = [lgw0, lgw1]
    sidx = [sidx0, sidx1]
    isem = [isem0, isem1, isem2, isem3]
    gsem = [gsem0, gsem1]
    lsem = [lsem0, lsem1]

    zrow = sid * RPT
    def _z(i, _):
        for k in range(8):
            hcol0[i, pl.ds(k * 16, 16)] = jnp.zeros((16,), jnp.float32)
        return 0
    lax.fori_loop(0, C2, _z, 0)
    nfull = RPT // C2
    def _zs(i, _):
        pltpu.sync_copy(hcol0, agg_sp.at[pl.ds(zrow + i * C2, C2)])
        return 0
    lax.fori_loop(0, nfull, _zs, 0)
    rem = RPT - nfull * C2
    if rem:
        pltpu.sync_copy(hcol0.at[pl.ds(0, rem)],
                        agg_sp.at[pl.ds(zrow + nfull * C2, rem)])
    pltpu.sync_copy(w2c_h, w2f)
    pltpu.sync_copy(sb2v_h, sb2b)
    plsc.subcore_barrier()

    def fire_idx(c, sl):
        gb = (w * NCH2 + c) * C2
        pltpu.async_copy(rowp.at[pl.ds(gb, C2)], ridx[sl], isem[sl])
        pltpu.async_copy(colp.at[pl.ds(gb, C2)], cidx[sl], isem[sl])
        pltpu.async_copy(tpad.at[pl.ds(gb, C2)], tbuf[sl], isem[sl])

    def wait_idx(c, sl):
        gb = (w * NCH2 + c) * C2
        pltpu.make_async_copy(rowp.at[pl.ds(gb, C2)], ridx[sl], isem[sl]).wait()
        pltpu.make_async_copy(colp.at[pl.ds(gb, C2)], cidx[sl], isem[sl]).wait()
        pltpu.make_async_copy(tpad.at[pl.ds(gb, C2)], tbuf[sl], isem[sl]).wait()

    def fire_gather(sd, si):
        pltpu.async_copy(abuf_h.at[ridx[si]], arow[sd], gsem[sd])
        pltpu.async_copy(bbuf_h.at[cidx[si]], bcol[sd], gsem[sd])
        pltpu.async_copy(hb_h.at[cidx[si]], hcol[sd], gsem[sd])

    def wait_gather(sd, si):
        pltpu.make_async_copy(abuf_h.at[ridx[si]], arow[sd], gsem[sd]).wait()
        pltpu.make_async_copy(bbuf_h.at[cidx[si]], bcol[sd], gsem[sd]).wait()
        pltpu.make_async_copy(hb_h.at[cidx[si]], hcol[sd], gsem[sd]).wait()

    def fire_scatter(c, sd):
        base = (w * NCH2 + c) * (2 * C2)
        pltpu.async_copy(lgw[sd], lgw_out.at[pl.ds(base, 2 * C2)], lsem[sd])
        pltpu.sync_copy(hcol[sd], agg_sp.at[sidx[sd]], add=True)

    def wait_lgw(sd):
        pltpu.make_async_copy(lgw_out.at[pl.ds(0, 2 * C2)], lgw[sd], lsem[sd]).wait()

    iota = lax.iota(jnp.int32, 16)

    def compute(sd, si):
        ar, bc = arow[sd], bcol[sd]
        lb, sx = lgw[sd], sidx[sd]
        tb, rx = tbuf[si], ridx[si]
        sb2 = sb2b[...]
        def group(j, _2):
            base16 = j * 16
            eidx = base16 + iota
            def kbody(k, acc):
                kv = jnp.full((16,), k, jnp.int32)
                av = plsc.load_gather(ar, [eidx, kv])
                bv = plsc.load_gather(bc, [eidx, kv])
                w2v = plsc.load_gather(w2f, [kv])
                return acc + jnp.maximum(av + bv, 0.0) * w2v
            acc = lax.fori_loop(0, 64, kbody, sb2, unroll=8)
            tv = tb[pl.ds(base16, 16)]
            rv = rx[pl.ds(base16, 16)]
            m = acc > tv
            lb[pl.ds(base16, 16)] = acc
            lb[pl.ds(C2 + base16, 16)] = jnp.where(m, 1.0, 0.0)
            sx[pl.ds(base16, 16)] = jnp.where(m, rv, ND)
            return 0
        lax.fori_loop(0, C2 // 16, group, 0)

    def stage(c, r, wl, fi, fg):
        if wl:
            wait_lgw(r % 2)
        if fi:
            fire_idx(c + 2, (r + 2) % 4)
        if fg:
            wait_idx(c + 1, (r + 1) % 4)
            fire_gather((r + 1) % 2, (r + 1) % 4)
        wait_gather(r % 2, r % 4)
        compute(r % 2, r % 4)
        fire_scatter(c, r % 2)

    fire_idx(0, 0)
    fire_idx(1, 1)
    wait_idx(0, 0)
    fire_gather(0, 0)
    stage(0, 0, False, True, True)
    stage(1, 1, False, True, True)
    def steady(it, _):
        for j in range(4):
            stage(2 + it * 4 + j, (2 + j) % 4, True, True, True)
        return 0
    lax.fori_loop(0, (NCH2 - 5) // 4, steady, 0)
    stage(NCH2 - 3, (NCH2 - 3) % 4, True, True, True)
    stage(NCH2 - 2, (NCH2 - 2) % 4, True, False, True)
    stage(NCH2 - 1, (NCH2 - 1) % 4, True, False, False)
    wait_lgw((NCH2 - 2) % 2)
    wait_lgw((NCH2 - 1) % 2)
    plsc.subcore_barrier()

    pltpu.sync_copy(agg_sp.at[pl.ds(zrow, RPT)], aggp.at[cid, pl.ds(zrow, RPT)])


def _scorer_pass(rowp, colp, tpad, abuf, bbuf, hb, w2c, sb2v):
    f = pl.kernel(
        _scorer_body,
        out_type=(
            jax.ShapeDtypeStruct((NW * NCH2 * 2 * C2,), jnp.float32),
            jax.ShapeDtypeStruct((NC, NP, TH), jnp.float32),
        ),
        mesh=_MESH,
        scratch_types=(
            *([pltpu.VMEM((C2,), jnp.int32)] * 8),
            *([pltpu.VMEM((C2,), jnp.float32)] * 4),
            *([pltpu.VMEM((C2, 64), jnp.float32)] * 4),
            *([pltpu.VMEM((C2, TH), jnp.float32)] * 2),
            *([pltpu.VMEM((2 * C2,), jnp.float32)] * 2),
            *([pltpu.VMEM((C2,), jnp.int32)] * 2),
            pltpu.VMEM((64,), jnp.float32),
            pltpu.VMEM((16,), jnp.float32),
            pltpu.VMEM_SHARED((NP, TH), jnp.float32),
            *([pltpu.SemaphoreType.DMA] * 8),
        ),
        compiler_params=_SC_PARAMS,
    )
    return f(rowp, colp, tpad, abuf, bbuf, hb, w2c, sb2v)


def _kernel_impl(x, edge_index, params, consts):
    p = params
    row = edge_index[0]
    col = edge_index[1]
    (a1cat, a2cat, em, s1, t1, s2, t2, cs, ct, w2p, b2p, tfix) = consts

    sl = jnp.arange(N, dtype=jnp.int32)
    padE1 = jnp.full((EP1 - ESL,), ND, jnp.int32)
    rr = lambda a: a.reshape(EPW1, NW).T.reshape(-1)
    rowsl = rr(jnp.concatenate([row, sl, padE1]))
    colsl = rr(jnp.concatenate([col, sl, padE1]))
    rowp = row
    colp = col
    tpad = tfix

    xpad = jnp.zeros((NP, D_IN), jnp.float32).at[:N].set(x)

    rb2 = p["res_b"].reshape(1, TH)
    g1b = p["g1_b"].reshape(1, TH)
    g2b = p["g2_b"].reshape(1, TH)

    xws1, d2a = _tc_call(
        _k1_body,
        [_row_spec(_RB, D_IN), _full_spec((D_IN, TH)), _full_spec((1, TH)),
         _full_spec((TH, TH)), _full_spec((TH, 32))],
        [_row_spec(_RB, TW), _row_spec(_RB, 16)],
        [jax.ShapeDtypeStruct((NP, TW), jnp.float32),
         jax.ShapeDtypeStruct((NP, 16), jnp.float32)],
        [xpad, p["res_W"], rb2, p["g1_W"], a1cat],
    )

    outp1 = _gat_edge_pass(rowsl, colsl, xws1, d2a)

    xws2, d2b = _tc_call(
        _k2_body,
        [pl.BlockSpec((NC, _RB, TW), lambda i: (0, i, 0)),
         _full_spec((16, TH)), _full_spec((1, TH)), _full_spec((1, TH)),
         _full_spec((1, TH)), _full_spec((TH, TH)), _full_spec((TH, 32))],
        [_row_spec(_RB, TW), _row_spec(_RB, 16)],
        [jax.ShapeDtypeStruct((NP, TW), jnp.float32),
         jax.ShapeDtypeStruct((NP, 16), jnp.float32)],
        [outp1, em, g1b, s1.reshape(1, TH), t1.reshape(1, TH),
         p["g2_W"], a2cat],
    )

    outp2 = _gat_edge_pass(rowsl, colsl, xws2, d2b)

    hb, abuf, bbuf = _tc_call(
        _k3_body,
        [pl.BlockSpec((NC, _RB, TW), lambda i: (0, i, 0)),
         _full_spec((16, TH)), _full_spec((1, TH)), _full_spec((1, TH)),
         _full_spec((1, TH)), _full_spec((TH, 64)), _full_spec((TH, 64)),
         _full_spec((1, 64))],
        [_row_spec(_RB, TH), _row_spec(_RB, 64), _row_spec(_RB, 64)],
        [jax.ShapeDtypeStruct((NP, TH), jnp.float32),
         jax.ShapeDtypeStruct((NP, 64), jnp.float32),
         jax.ShapeDtypeStruct((NP, 64), jnp.float32)],
        [outp2, em, g2b, s2.reshape(1, TH), t2.reshape(1, TH),
         p["s_W1"][:TH], p["s_W1"][TH:], p["s_b1"].reshape(1, 64)],
    )

    w2c = p["s_W2"][:, 0]
    sb2v = jnp.full((16,), p["s_b2"][0], jnp.float32)
    lgw, aggp = _scorer_pass(rowp, colp, tpad, abuf, bbuf, hb, w2c, sb2v)

    (out,) = _tc_call(
        _k4_body,
        [_row_spec(_RB, TH),
         pl.BlockSpec((NC, _RB, TH), lambda i: (0, i, 0)),
         _full_spec((TH, 64)), _full_spec((1, 64)), _full_spec((1, 64)),
         _full_spec((1, 64)), _full_spec((64, TH)), _full_spec((1, TH))],
        [_row_spec(_RB, TH)],
        [jax.ShapeDtypeStruct((NP, TH), jnp.float32)],
        [hb, aggp, p["c_W1"], p["c_b1"].reshape(1, 64), cs.reshape(1, 64),
         ct.reshape(1, 64), w2p, b2p.reshape(1, TH)],
    )

    lgw2 = lgw.reshape(NW * NCH2, 2, C2)
    logits = lgw2[:, 0, :].reshape(EP2)[:E]
    weights = lgw2[:, 1, :].reshape(EP2)[:E]
    return out[:N, :OUT], weights, logits


def _make_consts(params):
    p = params

    def acat(a_s, a_d):
        eye = jnp.eye(HEADS, dtype=jnp.float32)
        ms = (a_s[:, :, None] * eye[:, None, :]).reshape(TH, HEADS)
        md = (a_d[:, :, None] * eye[:, None, :]).reshape(TH, HEADS)
        return jnp.concatenate([ms, ms, md, md], axis=1)

    a1cat = acat(p["g1_as"], p["g1_ad"])
    a2cat = acat(p["g2_as"], p["g2_ad"])
    em = jnp.concatenate(
        [jnp.kron(jnp.eye(HEADS, dtype=jnp.float32), jnp.ones((1, 16), jnp.float32)),
         jnp.zeros((8, TH), jnp.float32)], axis=0)

    def bnst(g, b, m, v):
        s = g / jnp.sqrt(v + 1e-5)
        return s, b - m * s

    s1, t1 = bnst(p["bn1_g"], p["bn1_b"], p["bn1_m"], p["bn1_v"])
    s2, t2 = bnst(p["bn2_g"], p["bn2_b"], p["bn2_m"], p["bn2_v"])
    cs, ct = bnst(p["cbn_g"], p["cbn_b"], p["cbn_m"], p["cbn_v"])

    w2p = jnp.zeros((64, TH), jnp.float32).at[:, :OUT].set(p["c_W2"])
    b2p = jnp.full((TH,), -1e30, jnp.float32).at[:OUT].set(p["c_b2"])

    u = jax.random.uniform(jax.random.key(42), (E, 2),
                           minval=1e-6, maxval=1.0 - 1e-6)
    g = -jnp.log(-jnp.log(u))
    tfix = g[:, 0] - g[:, 1]

    return (a1cat, a2cat, em, s1, t1, s2, t2, cs, ct, w2p, b2p, tfix)


@jax.jit
def kernel(x, edge_index, params):
    consts = _make_consts(params)
    return _kernel_impl(x, edge_index.astype(jnp.int32), params, consts)
